# baseline (speedup 1.0000x reference)
import jax
import jax.numpy as jnp
from jax.experimental import pallas as pl


_BLK_B = 16


def _add_pos_kernel(x_ref, pos_ref, o_ref):
    o_ref[...] = x_ref[...] + pos_ref[0:128, :][None, :, :]


def kernel(x, pos_emb):
    B, L, D = x.shape
    grid = (B // _BLK_B,)
    return pl.pallas_call(
        _add_pos_kernel,
        grid=grid,
        in_specs=[
            pl.BlockSpec((_BLK_B, L, D), lambda i: (i, 0, 0)),
            pl.BlockSpec(pos_emb.shape, lambda i: (0, 0)),
        ],
        out_specs=pl.BlockSpec((_BLK_B, L, D), lambda i: (i, 0, 0)),
        out_shape=jax.ShapeDtypeStruct((B, L, D), x.dtype),
    )(x, pos_emb)


# speedup vs baseline: 1.5253x; 1.5253x over previous
import jax
import jax.numpy as jnp
from jax.experimental import pallas as pl


_BLK_B = 64


def _add_pos_kernel(x_ref, pos_ref, o_ref):
    o_ref[...] = x_ref[...] + pos_ref[0:128, :][None, :, :]


def kernel(x, pos_emb):
    B, L, D = x.shape
    grid = (B // _BLK_B,)
    return pl.pallas_call(
        _add_pos_kernel,
        grid=grid,
        in_specs=[
            pl.BlockSpec((_BLK_B, L, D), lambda i: (i, 0, 0)),
            pl.BlockSpec(pos_emb.shape, lambda i: (0, 0)),
        ],
        out_specs=pl.BlockSpec((_BLK_B, L, D), lambda i: (i, 0, 0)),
        out_shape=jax.ShapeDtypeStruct((B, L, D), x.dtype),
    )(x, pos_emb)


# speedup vs baseline: 1.5436x; 1.0120x over previous
import jax
import jax.numpy as jnp
from jax.experimental import pallas as pl


_BLK_B = 128


def _add_pos_kernel(x_ref, pos_ref, o_ref):
    o_ref[...] = x_ref[...] + pos_ref[0:128, :][None, :, :]


def kernel(x, pos_emb):
    B, L, D = x.shape
    grid = (B // _BLK_B,)
    return pl.pallas_call(
        _add_pos_kernel,
        grid=grid,
        in_specs=[
            pl.BlockSpec((_BLK_B, L, D), lambda i: (i, 0, 0)),
            pl.BlockSpec(pos_emb.shape, lambda i: (0, 0)),
        ],
        out_specs=pl.BlockSpec((_BLK_B, L, D), lambda i: (i, 0, 0)),
        out_shape=jax.ShapeDtypeStruct((B, L, D), x.dtype),
    )(x, pos_emb)
